# initial kernel scaffold (unmeasured)
import jax
import jax.numpy as jnp
from jax import lax
from jax.experimental import pallas as pl
from jax.experimental.pallas import tpu as pltpu


def kernel(
    x,
):
    def body(*refs):
        pass

    out_shape = jax.ShapeDtypeStruct(..., jnp.float32)
    return pl.pallas_call(body, out_shape=out_shape)(...)


# baseline (device time: 103215 ns/iter reference)
import jax
import jax.numpy as jnp
from jax import lax
from jax.experimental import pallas as pl
from jax.experimental.pallas import tpu as pltpu

Y = 4


def kernel(x):
    x = x.astype(jnp.bfloat16)
    m, n = x.shape
    n_out = n // Y

    def body(x_ref, out_ref, send_sems, recv_sems):
        mx = lax.axis_index("x")
        my = lax.axis_index("y")
        mz = lax.axis_index("z")

        barrier = pltpu.get_barrier_semaphore()
        for d in range(1, Y):
            k = (my + d) % Y
            pl.semaphore_signal(
                barrier, inc=1,
                device_id=(mx, k, mz), device_id_type=pl.DeviceIdType.MESH,
            )
        pl.semaphore_wait(barrier, Y - 1)

        rdmas = []
        for d in range(1, Y):
            k = (my + d) % Y
            rdma = pltpu.make_async_remote_copy(
                src_ref=x_ref.at[:, pl.ds(k * n_out, n_out)],
                dst_ref=out_ref.at[pl.ds(my * m, m), :],
                send_sem=send_sems.at[d - 1],
                recv_sem=recv_sems.at[d - 1],
                device_id=(mx, k, mz),
                device_id_type=pl.DeviceIdType.MESH,
            )
            rdma.start()
            rdmas.append(rdma)

        out_ref[pl.ds(my * m, m), :] = x_ref[:, pl.ds(my * n_out, n_out)]

        for rdma in rdmas:
            rdma.wait_send()
            rdma.wait_recv()

    return pl.pallas_call(
        body,
        out_shape=jax.ShapeDtypeStruct((Y * m, n_out), x.dtype),
        in_specs=[pl.BlockSpec(memory_space=pltpu.VMEM)],
        out_specs=pl.BlockSpec(memory_space=pltpu.VMEM),
        scratch_shapes=[
            pltpu.SemaphoreType.DMA((Y - 1,)),
            pltpu.SemaphoreType.DMA((Y - 1,)),
        ],
        compiler_params=pltpu.CompilerParams(collective_id=0),
    )(x)


# device time: 102517 ns/iter; 1.0068x vs baseline; 1.0068x over previous
import jax
import jax.numpy as jnp
from jax import lax
from jax.experimental import pallas as pl
from jax.experimental.pallas import tpu as pltpu

Y = 4


def kernel(x):
    m, n = x.shape
    n_out = n // Y

    def body(x_ref, out_ref, xbf_ref, send_sems, recv_sems):
        mx = lax.axis_index("x")
        my = lax.axis_index("y")
        mz = lax.axis_index("z")

        barrier = pltpu.get_barrier_semaphore()
        for d in range(1, Y):
            k = (my + d) % Y
            pl.semaphore_signal(
                barrier, inc=1,
                device_id=(mx, k, mz), device_id_type=pl.DeviceIdType.MESH,
            )
        xbf_ref[:, :] = x_ref[:, :].astype(jnp.bfloat16)
        pl.semaphore_wait(barrier, Y - 1)

        rdmas = []
        for d in range(1, Y):
            k = (my + d) % Y
            rdma = pltpu.make_async_remote_copy(
                src_ref=xbf_ref.at[:, pl.ds(k * n_out, n_out)],
                dst_ref=out_ref.at[pl.ds(my * m, m), :],
                send_sem=send_sems.at[d - 1],
                recv_sem=recv_sems.at[d - 1],
                device_id=(mx, k, mz),
                device_id_type=pl.DeviceIdType.MESH,
            )
            rdma.start()
            rdmas.append(rdma)

        out_ref[pl.ds(my * m, m), :] = xbf_ref[:, pl.ds(my * n_out, n_out)]

        for rdma in rdmas:
            rdma.wait_send()
            rdma.wait_recv()

    return pl.pallas_call(
        body,
        out_shape=jax.ShapeDtypeStruct((Y * m, n_out), jnp.bfloat16),
        in_specs=[pl.BlockSpec(memory_space=pltpu.VMEM)],
        out_specs=pl.BlockSpec(memory_space=pltpu.VMEM),
        scratch_shapes=[
            pltpu.VMEM((m, n), jnp.bfloat16),
            pltpu.SemaphoreType.DMA((Y - 1,)),
            pltpu.SemaphoreType.DMA((Y - 1,)),
        ],
        compiler_params=pltpu.CompilerParams(collective_id=0),
    )(x)
